# Initial kernel scaffold; baseline (speedup 1.0000x reference)
import jax
import jax.numpy as jnp
from jax.experimental import pallas as pl


def kernel(points, features, batch, params):
    raise NotImplementedError("write your pallas kernel here")


# speedup vs baseline: 1.0543x; 1.0543x over previous
import jax
import jax.numpy as jnp
from jax.experimental import pallas as pl

_DIMS = [64, 128, 256, 512, 512, 512]
_GRID = [0.02, 0.06, 0.15, 0.375, 0.9375]
_NB = 4


def _relu_copy_body(x_ref, o_ref):
    o_ref[...] = jnp.maximum(x_ref[...], 0.0)


def _valid_masks(points, batch):
    P = points.shape[0]
    row = jnp.arange(P)
    b = batch.astype(jnp.int32)
    prev = jnp.ones((P,), jnp.bool_)
    valids = []
    for gs in _GRID:
        v = jnp.floor(points / gs).astype(jnp.int32)
        key = ((b * 256 + v[:, 0]) * 256 + v[:, 1]) * 256 + v[:, 2]
        same = key[:, None] == key[None, :]
        earlier = same & prev[None, :] & (row[None, :] < row[:, None])
        valid = prev & ~jnp.any(earlier, axis=1)
        valids.append(valid)
        prev = valid
    return valids


def kernel(points, features, batch, params):
    p = params
    b = batch.astype(jnp.int32)
    valids = _valid_masks(points, batch)

    n2 = jnp.sum(points * points, axis=1)
    d = n2[:, None] + n2[None, :] - 2.0 * (points @ points.T)

    def topk(dm):
        _, idx = jax.lax.top_k(-dm, 16)
        return idx

    same_b = b[:, None] == b[None, :]
    es = [topk(d)]
    ef = []
    prev = jnp.ones((8192,), jnp.bool_)
    for i in range(5):
        ef.append(topk(jnp.where(same_b & prev[None, :], d, jnp.inf)))
        es.append(topk(jnp.where(valids[i][None, :], d, jnp.inf)))
        prev = valids[i]

    x = jax.nn.relu(features @ p['W_embed'] + p['b_embed'])

    def conv(x, nbr, W, bias):
        C = x.shape[1]
        tab = jnp.concatenate([x, points], axis=1)
        g = jnp.mean(tab[nbr], axis=1)
        g = g.at[:, C:].add(-points)
        return jax.nn.relu(g @ W + bias)

    x = conv(x, es[0], p['W_self0'], p['b_self0'])
    for i in range(5):
        x = conv(x, ef[i], p['W_fwd' + str(i)], p['b_fwd' + str(i)])
        x = conv(x, es[i + 1], p['W_self' + str(i + 1)], p['b_self' + str(i + 1)])

    fb = jnp.where(valids[-1], b, _NB)
    sums = jax.ops.segment_sum(x, fb, num_segments=_NB)
    cnts = jax.ops.segment_sum(jnp.ones((8192,), jnp.float32), fb, num_segments=_NB)
    out = sums / jnp.maximum(cnts, 1.0)[:, None]

    out = pl.pallas_call(
        _relu_copy_body,
        out_shape=jax.ShapeDtypeStruct(out.shape, out.dtype),
    )(out) + jnp.minimum(out, 0.0)
    return out


# speedup vs baseline: 4.0792x; 3.8691x over previous
import functools
import jax
import jax.numpy as jnp
from jax import lax
from jax.experimental import pallas as pl
from jax.experimental.pallas import tpu as pltpu
from jax.experimental.pallas import tpu_sc as plsc

_DIMS = [64, 128, 256, 512, 512, 512]
_GRID_SIZES = [0.02, 0.06, 0.15, 0.375, 0.9375]
_P = 8192
_NB = 4
_QB = 128
_NSEL = 11


def _pad128(n):
    return (n + 127) // 128 * 128


def _voxel_key(pts, b32, gs):
    v = jnp.floor(pts / gs).astype(jnp.int32)
    return (((b32[:, 0] * 256 + v[:, 0]) * 256 + v[:, 1]) * 256 + v[:, 2])[:, None]


def _mask_body(gs, pts_ref, b_ref, ptsr_ref, br_ref, vr_ref, vc_ref, out_ref):
    i = pl.program_id(0)
    key = _voxel_key(pts_ref[...], b_ref[...], gs)
    kr = _voxel_key(ptsr_ref[...], br_ref[...], gs)
    vr = vr_ref[...]
    kc = key.reshape(1, _P)
    rowi = i * 256 + lax.broadcasted_iota(jnp.int32, (256, 1), 0)
    acc = jnp.zeros((256, 1), jnp.int32)
    for cc in range(4):
        kcc = kc[:, cc * 2048:(cc + 1) * 2048]
        vcc = vc_ref[:, pl.dslice(cc * 2048, 2048)]
        coli = cc * 2048 + lax.broadcasted_iota(jnp.int32, (1, 2048), 1)
        e = (kr == kcc) & (vcc != 0) & (coli < rowi)
        acc = acc | jnp.any(e, axis=1, keepdims=True).astype(jnp.int32)
    out_ref[...] = vr * (1 - acc)


def _valid_level(gs, points, b_row, prev_row, prev_col):
    return pl.pallas_call(
        functools.partial(_mask_body, gs),
        grid=(32,),
        in_specs=[
            pl.BlockSpec((_P, 3), lambda i: (0, 0)),
            pl.BlockSpec((_P, 1), lambda i: (0, 0)),
            pl.BlockSpec((256, 3), lambda i: (i, 0)),
            pl.BlockSpec((256, 1), lambda i: (i, 0)),
            pl.BlockSpec((256, 1), lambda i: (i, 0)),
            pl.BlockSpec((1, _P), lambda i: (0, 0)),
        ],
        out_specs=pl.BlockSpec((256, 1), lambda i: (i, 0)),
        out_shape=jax.ShapeDtypeStruct((_P, 1), jnp.int32),
    )(points, b_row, points, b_row, prev_row, prev_col)


def _sel_body(pts_ref, ptsq_ref, bc_ref, bq_ref, vall_ref, out_ref, dm_ref):
    pts = pts_ref[...]
    ptsq = ptsq_ref[...]
    n2c = jnp.sum(pts * pts, axis=1, keepdims=True)
    n2q = jnp.sum(ptsq * ptsq, axis=1)[None, :]
    d = n2c + n2q - 2.0 * lax.dot_general(
        pts, ptsq, (((1,), (1,)), ((), ())),
        preferred_element_type=jnp.float32)
    sameb = bc_ref[...] == bq_ref[...]
    inf = jnp.float32(jnp.inf)

    def extract(sel_slot):
        def body(t, acc):
            dm = dm_ref[...]
            a = jnp.argmin(dm, axis=0).astype(jnp.int32)
            ri = lax.broadcasted_iota(jnp.int32, (_P, _QB), 0)
            dm_ref[...] = jnp.where(ri == a[None, :], inf, dm)
            ti = lax.broadcasted_iota(jnp.int32, (16, _QB), 0)
            return jnp.where(ti == t, a[None, :], acc)

        acc = lax.fori_loop(0, 16, body, jnp.zeros((16, _QB), jnp.int32))
        out_ref[sel_slot * 16:(sel_slot + 1) * 16, :] = acc

    dm_ref[...] = d
    extract(0)
    for i in range(5):
        if i == 0:
            m = sameb
        else:
            m = sameb & (vall_ref[:, pl.dslice(i - 1, 1)] != 0)
        dm_ref[...] = jnp.where(m, d, inf)
        extract(1 + 2 * i)
        v = vall_ref[:, pl.dslice(i, 1)] != 0
        dm_ref[...] = jnp.where(v, d, inf)
        extract(2 + 2 * i)


def _selection(points, b_row, b_col, valid_cols):
    return pl.pallas_call(
        _sel_body,
        grid=(_P // _QB,),
        in_specs=[
            pl.BlockSpec((_P, 3), lambda i: (0, 0)),
            pl.BlockSpec((_QB, 3), lambda i: (i, 0)),
            pl.BlockSpec((_P, 1), lambda i: (0, 0)),
            pl.BlockSpec((1, _QB), lambda i: (0, i)),
            pl.BlockSpec((_P, 8), lambda i: (0, 0)),
        ],
        out_specs=pl.BlockSpec((_NSEL * 16, _QB), lambda i: (0, i)),
        out_shape=jax.ShapeDtypeStruct((_NSEL * 16, _P), jnp.int32),
        scratch_shapes=[pltpu.VMEM((_P, _QB), jnp.float32)],
    )(points, points, b_row, b_col, valid_cols)


def _gather_sum(table, idx):
    Dp = table.shape[1]
    info = plsc.get_sparse_core_info()
    nw = info.num_cores * info.num_subcores
    bpw = _P // nw
    mesh = plsc.VectorSubcoreMesh(core_axis_name="c", subcore_axis_name="s")

    @functools.partial(
        pl.kernel, mesh=mesh,
        out_type=jax.ShapeDtypeStruct((_P, Dp), jnp.float32),
        scratch_types=[
            pltpu.VMEM((bpw, 16), jnp.int32),
            pltpu.VMEM((16, Dp), jnp.float32),
            pltpu.VMEM((8, Dp), jnp.float32),
            pltpu.SemaphoreType.DMA,
        ],
    )
    def k(table_hbm, idx_hbm, out_hbm, idx_v, rows_v, out_v, sem):
        wid = lax.axis_index("s") * info.num_cores + lax.axis_index("c")
        base = wid * bpw
        pltpu.sync_copy(idx_hbm.at[pl.dslice(base, bpw)], idx_v)

        def point(p, carry):
            cp = pltpu.make_async_copy(table_hbm.at[idx_v.at[p]], rows_v, sem)
            cp.start()
            cp.wait()
            r = lax.rem(p, 8)
            for c in range(Dp // 16):
                s = rows_v[0, pl.dslice(c * 16, 16)]
                for rr in range(1, 16):
                    s = s + rows_v[rr, pl.dslice(c * 16, 16)]
                out_v[r, pl.dslice(c * 16, 16)] = s

            @pl.when(r == 7)
            def _():
                off = pl.multiple_of(base + p - 7, 8)
                pltpu.sync_copy(out_v, out_hbm.at[pl.dslice(off, 8)])
            return carry

        lax.fori_loop(0, bpw, point, 0)

    return k(table, idx)


def _table_mm(g, w, bvec, pts, w3):
    Din = g.shape[1]
    out_w = w.shape[1]

    def body(g_ref, w_ref, b_ref, p_ref, w3_ref, out_ref):
        h = lax.dot_general(g_ref[...], w_ref[...], (((1,), (0,)), ((), ())),
                            preferred_element_type=jnp.float32)
        adj = lax.dot_general(p_ref[...], w3_ref[...], (((1,), (0,)), ((), ())),
                              preferred_element_type=jnp.float32)
        out_ref[...] = jnp.maximum(h + b_ref[...] - adj, 0.0)

    return pl.pallas_call(
        body,
        grid=(_P // 512,),
        in_specs=[
            pl.BlockSpec((512, Din), lambda i: (i, 0)),
            pl.BlockSpec((Din, out_w), lambda i: (0, 0)),
            pl.BlockSpec((1, out_w), lambda i: (0, 0)),
            pl.BlockSpec((512, 3), lambda i: (i, 0)),
            pl.BlockSpec((3, out_w), lambda i: (0, 0)),
        ],
        out_specs=pl.BlockSpec((512, out_w), lambda i: (i, 0)),
        out_shape=jax.ShapeDtypeStruct((_P, out_w), jnp.float32),
    )(g, w, bvec, pts, w3)


def _pool(x, b_row, v_row):
    def body(x_ref, b_ref, v_ref, out_ref):
        si = lax.broadcasted_iota(jnp.int32, (_P, 128), 1)
        oh = ((b_ref[...] == si) & (v_ref[...] != 0)).astype(jnp.float32)
        sums = lax.dot_general(oh, x_ref[...], (((0,), (0,)), ((), ())),
                               preferred_element_type=jnp.float32)
        cnts = jnp.sum(oh, axis=0)[:, None]
        out_ref[...] = (sums / jnp.maximum(cnts, 1.0))[0:4, :]

    return pl.pallas_call(
        body,
        out_shape=jax.ShapeDtypeStruct((4, 512), jnp.float32),
    )(x, b_row, v_row)


def kernel(points, features, batch, params):
    p = params
    b32 = batch.astype(jnp.int32)
    b_row = b32[:, None]
    b_col = b32[None, :]

    valid_rows = []
    prev_row = jnp.ones((_P, 1), jnp.int32)
    prev_col = jnp.ones((1, _P), jnp.int32)
    for gs in _GRID_SIZES:
        vr = _valid_level(gs, points, b_row, prev_row, prev_col)
        valid_rows.append(vr)
        prev_row = vr
        prev_col = vr.reshape(1, _P)
    valid_cols = jnp.concatenate(
        valid_rows + [jnp.zeros((_P, 3), jnp.int32)], axis=1)

    sel = _selection(points, b_row, b_col, valid_cols)
    idxs = [sel[s * 16:(s + 1) * 16, :].T for s in range(_NSEL)]

    w_emb = jnp.zeros((3, _pad128(_DIMS[0] + 3)), jnp.float32)
    w_emb = w_emb.at[:, :_DIMS[0]].set(p['W_embed'])
    b_emb = jnp.zeros((1, _pad128(_DIMS[0] + 3)), jnp.float32)
    b_emb = b_emb.at[0, :_DIMS[0]].set(p['b_embed'])
    z3 = jnp.zeros((3, _pad128(_DIMS[0] + 3)), jnp.float32)
    table = _table_mm(features, w_emb, b_emb, points, z3)
    table = table.at[:, _DIMS[0]:_DIMS[0] + 3].set(points)

    def conv(tab, nbr, W, bias, cout, last):
        Din = tab.shape[1]
        g = _gather_sum(tab, nbr)
        out_w = cout if last else _pad128(cout + 3)
        Wfull = jnp.zeros((Din, out_w), jnp.float32)
        Wfull = Wfull.at[:W.shape[0], :cout].set(W / 16.0)
        bfull = jnp.zeros((1, out_w), jnp.float32)
        bfull = bfull.at[0, :cout].set(bias)
        w3full = jnp.zeros((3, out_w), jnp.float32)
        w3full = w3full.at[:, :cout].set(W[W.shape[0] - 3:, :])
        out = _table_mm(g, Wfull, bfull, points, w3full)
        if not last:
            out = out.at[:, cout:cout + 3].set(points)
        return out

    x = conv(table, idxs[0], p['W_self0'], p['b_self0'], _DIMS[0], False)
    for i in range(5):
        x = conv(x, idxs[1 + 2 * i], p['W_fwd' + str(i)], p['b_fwd' + str(i)],
                 _DIMS[i + 1], False)
        x = conv(x, idxs[2 + 2 * i], p['W_self' + str(i + 1)],
                 p['b_self' + str(i + 1)], _DIMS[i + 1], i == 4)

    return _pool(x, b_row, valid_rows[4])


# speedup vs baseline: 5.2635x; 1.2903x over previous
import functools
import jax
import jax.numpy as jnp
from jax import lax
from jax.experimental import pallas as pl
from jax.experimental.pallas import tpu as pltpu
from jax.experimental.pallas import tpu_sc as plsc

_DIMS = [64, 128, 256, 512, 512, 512]
_GRID_SIZES = [0.02, 0.06, 0.15, 0.375, 0.9375]
_P = 8192
_NB = 4
_QB = 128
_NSEL = 11


def _pad128(n):
    return (n + 127) // 128 * 128


def _voxel_key(pts, b32, gs):
    v = jnp.floor(pts / gs).astype(jnp.int32)
    return (((b32[:, 0] * 256 + v[:, 0]) * 256 + v[:, 1]) * 256 + v[:, 2])[:, None]


def _mask_body(gs, pts_ref, b_ref, ptsr_ref, br_ref, vr_ref, vc_ref, out_ref):
    i = pl.program_id(0)
    key = _voxel_key(pts_ref[...], b_ref[...], gs)
    kr = _voxel_key(ptsr_ref[...], br_ref[...], gs)
    vr = vr_ref[...]
    kc = key.reshape(1, _P)
    rowi = i * 256 + lax.broadcasted_iota(jnp.int32, (256, 1), 0)
    acc = jnp.zeros((256, 1), jnp.int32)
    for cc in range(4):
        kcc = kc[:, cc * 2048:(cc + 1) * 2048]
        vcc = vc_ref[:, pl.dslice(cc * 2048, 2048)]
        coli = cc * 2048 + lax.broadcasted_iota(jnp.int32, (1, 2048), 1)
        e = (kr == kcc) & (vcc != 0) & (coli < rowi)
        acc = acc | jnp.any(e, axis=1, keepdims=True).astype(jnp.int32)
    out_ref[...] = vr * (1 - acc)


def _valid_level(gs, points, b_row, prev_row, prev_col):
    return pl.pallas_call(
        functools.partial(_mask_body, gs),
        grid=(32,),
        in_specs=[
            pl.BlockSpec((_P, 3), lambda i: (0, 0)),
            pl.BlockSpec((_P, 1), lambda i: (0, 0)),
            pl.BlockSpec((256, 3), lambda i: (i, 0)),
            pl.BlockSpec((256, 1), lambda i: (i, 0)),
            pl.BlockSpec((256, 1), lambda i: (i, 0)),
            pl.BlockSpec((1, _P), lambda i: (0, 0)),
        ],
        out_specs=pl.BlockSpec((256, 1), lambda i: (i, 0)),
        out_shape=jax.ShapeDtypeStruct((_P, 1), jnp.int32),
    )(points, b_row, points, b_row, prev_row, prev_col)


def _sel_body(pts_ref, ptsq_ref, bc_ref, bq_ref, vall_ref, out_ref, dm_ref):
    pts = pts_ref[...]
    ptsq = ptsq_ref[...]
    n2c = jnp.sum(pts * pts, axis=1, keepdims=True)
    n2q = jnp.sum(ptsq * ptsq, axis=1)[None, :]
    d = n2c + n2q - 2.0 * lax.dot_general(
        pts, ptsq, (((1,), (1,)), ((), ())),
        preferred_element_type=jnp.float32)
    sameb = bc_ref[...] == bq_ref[...]
    inf = jnp.float32(jnp.inf)

    ri = lax.broadcasted_iota(jnp.int32, (_P, _QB), 0)

    def extract(sel_slot):
        def body(t, carry):
            a_prev, acc = carry
            dm = jnp.where(ri == a_prev[None, :], inf, dm_ref[...])
            dm_ref[...] = dm
            a = jnp.argmin(dm, axis=0).astype(jnp.int32)
            ti = lax.broadcasted_iota(jnp.int32, (16, _QB), 0)
            return a, jnp.where(ti == t, a[None, :], acc)

        _, acc = lax.fori_loop(
            0, 16, body,
            (jnp.full((_QB,), -1, jnp.int32), jnp.zeros((16, _QB), jnp.int32)))
        out_ref[sel_slot * 16:(sel_slot + 1) * 16, :] = acc

    dm_ref[...] = d
    extract(0)
    for i in range(5):
        if i == 0:
            m = sameb
        else:
            m = sameb & (vall_ref[:, pl.dslice(i - 1, 1)] != 0)
        dm_ref[...] = jnp.where(m, d, inf)
        extract(1 + 2 * i)
        v = vall_ref[:, pl.dslice(i, 1)] != 0
        dm_ref[...] = jnp.where(v, d, inf)
        extract(2 + 2 * i)


def _selection(points, b_row, b_col, valid_cols):
    return pl.pallas_call(
        _sel_body,
        grid=(_P // _QB,),
        in_specs=[
            pl.BlockSpec((_P, 3), lambda i: (0, 0)),
            pl.BlockSpec((_QB, 3), lambda i: (i, 0)),
            pl.BlockSpec((_P, 1), lambda i: (0, 0)),
            pl.BlockSpec((1, _QB), lambda i: (0, i)),
            pl.BlockSpec((_P, 8), lambda i: (0, 0)),
        ],
        out_specs=pl.BlockSpec((_NSEL * 16, _QB), lambda i: (0, i)),
        out_shape=jax.ShapeDtypeStruct((_NSEL * 16, _P), jnp.int32),
        scratch_shapes=[pltpu.VMEM((_P, _QB), jnp.float32)],
    )(points, points, b_row, b_col, valid_cols)


def _gather_sum(table, idx):
    Dp = table.shape[1]
    info = plsc.get_sparse_core_info()
    nw = info.num_cores * info.num_subcores
    bpw = _P // nw
    mesh = plsc.VectorSubcoreMesh(core_axis_name="c", subcore_axis_name="s")

    @functools.partial(
        pl.kernel, mesh=mesh,
        out_type=jax.ShapeDtypeStruct((_P, Dp), jnp.float32),
        scratch_types=[
            pltpu.VMEM((bpw, 16), jnp.int32),
            pltpu.VMEM((16, Dp), jnp.float32),
            pltpu.VMEM((16, Dp), jnp.float32),
            pltpu.VMEM((8, Dp), jnp.float32),
            pltpu.SemaphoreType.DMA,
            pltpu.SemaphoreType.DMA,
        ],
    )
    def k(table_hbm, idx_hbm, out_hbm, idx_v, rows0, rows1, out_v, sem0, sem1):
        wid = lax.axis_index("s") * info.num_cores + lax.axis_index("c")
        base = wid * bpw
        pltpu.sync_copy(idx_hbm.at[pl.dslice(base, bpw)], idx_v)
        pltpu.make_async_copy(table_hbm.at[idx_v.at[0]], rows0, sem0).start()

        def accum(rows_v, p):
            r = lax.rem(p, 8)
            for c in range(Dp // 16):
                s = rows_v[0, pl.dslice(c * 16, 16)]
                for rr in range(1, 16):
                    s = s + rows_v[rr, pl.dslice(c * 16, 16)]
                out_v[r, pl.dslice(c * 16, 16)] = s

            @pl.when(r == 7)
            def _():
                off = pl.multiple_of(base + p - 7, 8)
                pltpu.sync_copy(out_v, out_hbm.at[pl.dslice(off, 8)])

        def pair(i, carry):
            p0 = 2 * i
            p1 = 2 * i + 1
            pltpu.make_async_copy(table_hbm.at[idx_v.at[p1]], rows1, sem1).start()
            pltpu.make_async_copy(table_hbm.at[idx_v.at[p0]], rows0, sem0).wait()
            accum(rows0, p0)

            @pl.when(p1 + 1 < bpw)
            def _():
                pltpu.make_async_copy(
                    table_hbm.at[idx_v.at[p1 + 1]], rows0, sem0).start()
            pltpu.make_async_copy(table_hbm.at[idx_v.at[p1]], rows1, sem1).wait()
            accum(rows1, p1)
            return carry

        lax.fori_loop(0, bpw // 2, pair, 0)

    return k(table, idx)


def _table_mm(g, w, bvec, pts, w3):
    Din = g.shape[1]
    out_w = w.shape[1]

    def body(g_ref, w_ref, b_ref, p_ref, w3_ref, out_ref):
        h = lax.dot_general(g_ref[...], w_ref[...], (((1,), (0,)), ((), ())),
                            preferred_element_type=jnp.float32)
        adj = lax.dot_general(p_ref[...], w3_ref[...], (((1,), (0,)), ((), ())),
                              preferred_element_type=jnp.float32)
        out_ref[...] = jnp.maximum(h + b_ref[...] - adj, 0.0)

    return pl.pallas_call(
        body,
        grid=(_P // 512,),
        in_specs=[
            pl.BlockSpec((512, Din), lambda i: (i, 0)),
            pl.BlockSpec((Din, out_w), lambda i: (0, 0)),
            pl.BlockSpec((1, out_w), lambda i: (0, 0)),
            pl.BlockSpec((512, 3), lambda i: (i, 0)),
            pl.BlockSpec((3, out_w), lambda i: (0, 0)),
        ],
        out_specs=pl.BlockSpec((512, out_w), lambda i: (i, 0)),
        out_shape=jax.ShapeDtypeStruct((_P, out_w), jnp.float32),
    )(g, w, bvec, pts, w3)


def _pool(x, b_row, v_row):
    def body(x_ref, b_ref, v_ref, out_ref):
        si = lax.broadcasted_iota(jnp.int32, (_P, 128), 1)
        oh = ((b_ref[...] == si) & (v_ref[...] != 0)).astype(jnp.float32)
        sums = lax.dot_general(oh, x_ref[...], (((0,), (0,)), ((), ())),
                               preferred_element_type=jnp.float32)
        cnts = jnp.sum(oh, axis=0)[:, None]
        out_ref[...] = (sums / jnp.maximum(cnts, 1.0))[0:4, :]

    return pl.pallas_call(
        body,
        out_shape=jax.ShapeDtypeStruct((4, 512), jnp.float32),
    )(x, b_row, v_row)


def kernel(points, features, batch, params):
    p = params
    b32 = batch.astype(jnp.int32)
    b_row = b32[:, None]
    b_col = b32[None, :]

    valid_rows = []
    prev_row = jnp.ones((_P, 1), jnp.int32)
    prev_col = jnp.ones((1, _P), jnp.int32)
    for gs in _GRID_SIZES:
        vr = _valid_level(gs, points, b_row, prev_row, prev_col)
        valid_rows.append(vr)
        prev_row = vr
        prev_col = vr.reshape(1, _P)
    valid_cols = jnp.concatenate(
        valid_rows + [jnp.zeros((_P, 3), jnp.int32)], axis=1)

    sel = _selection(points, b_row, b_col, valid_cols)
    idxs = [sel[s * 16:(s + 1) * 16, :].T for s in range(_NSEL)]

    w_emb = jnp.zeros((3, _pad128(_DIMS[0] + 3)), jnp.float32)
    w_emb = w_emb.at[:, :_DIMS[0]].set(p['W_embed'])
    b_emb = jnp.zeros((1, _pad128(_DIMS[0] + 3)), jnp.float32)
    b_emb = b_emb.at[0, :_DIMS[0]].set(p['b_embed'])
    z3 = jnp.zeros((3, _pad128(_DIMS[0] + 3)), jnp.float32)
    table = _table_mm(features, w_emb, b_emb, points, z3)
    table = table.at[:, _DIMS[0]:_DIMS[0] + 3].set(points)

    def conv(tab, nbr, W, bias, cout, last):
        Din = tab.shape[1]
        g = _gather_sum(tab, nbr)
        out_w = cout if last else _pad128(cout + 3)
        Wfull = jnp.zeros((Din, out_w), jnp.float32)
        Wfull = Wfull.at[:W.shape[0], :cout].set(W / 16.0)
        bfull = jnp.zeros((1, out_w), jnp.float32)
        bfull = bfull.at[0, :cout].set(bias)
        w3full = jnp.zeros((3, out_w), jnp.float32)
        w3full = w3full.at[:, :cout].set(W[W.shape[0] - 3:, :])
        out = _table_mm(g, Wfull, bfull, points, w3full)
        if not last:
            out = out.at[:, cout:cout + 3].set(points)
        return out

    x = conv(table, idxs[0], p['W_self0'], p['b_self0'], _DIMS[0], False)
    for i in range(5):
        x = conv(x, idxs[1 + 2 * i], p['W_fwd' + str(i)], p['b_fwd' + str(i)],
                 _DIMS[i + 1], False)
        x = conv(x, idxs[2 + 2 * i], p['W_self' + str(i + 1)],
                 p['b_self' + str(i + 1)], _DIMS[i + 1], i == 4)

    return _pool(x, b_row, valid_rows[4])
